# baseline (device time: 55185 ns/iter reference)
import jax
import jax.numpy as jnp
from jax import lax
from jax.experimental import pallas as pl
from jax.experimental.pallas import tpu as pltpu


def kernel(x, pi):
    def body(pi_ref, x_ref, out_ref, send_buf, send_sem, recv_sem):
        my_x = lax.axis_index("x")
        my_y = lax.axis_index("y")
        my_z = lax.axis_index("z")
        tgt = pi_ref[my_x]

        barrier = pltpu.get_barrier_semaphore()

        @pl.when(tgt == my_x)
        def _():
            out_ref[...] = x_ref[...].astype(out_ref.dtype)

        @pl.when(tgt != my_x)
        def _():
            pl.semaphore_signal(
                barrier,
                inc=1,
                device_id=(tgt, my_y, my_z),
                device_id_type=pl.DeviceIdType.MESH,
            )
            pl.semaphore_wait(barrier, 1)

            send_buf[...] = x_ref[...].astype(send_buf.dtype)
            rdma = pltpu.make_async_remote_copy(
                src_ref=send_buf,
                dst_ref=out_ref,
                send_sem=send_sem,
                recv_sem=recv_sem,
                device_id=(tgt, my_y, my_z),
                device_id_type=pl.DeviceIdType.MESH,
            )
            rdma.start()
            rdma.wait()

    return pl.pallas_call(
        body,
        out_shape=jax.ShapeDtypeStruct(x.shape, jnp.bfloat16),
        in_specs=[
            pl.BlockSpec(memory_space=pltpu.SMEM),
            pl.BlockSpec(memory_space=pltpu.VMEM),
        ],
        out_specs=pl.BlockSpec(memory_space=pltpu.VMEM),
        scratch_shapes=[
            pltpu.VMEM(x.shape, jnp.bfloat16),
            pltpu.SemaphoreType.DMA,
            pltpu.SemaphoreType.DMA,
        ],
        compiler_params=pltpu.CompilerParams(collective_id=0),
    )(pi, x)


# device time: 54939 ns/iter; 1.0045x vs baseline; 1.0045x over previous
import jax
import jax.numpy as jnp
from jax import lax
from jax.experimental import pallas as pl
from jax.experimental.pallas import tpu as pltpu

_N_CHUNKS = 8


def kernel(x, pi):
    _, m, _ = x.shape
    rows = m // _N_CHUNKS

    def body(pi_ref, x_ref, out_ref, send_buf, send_sems, recv_sems):
        my_x = lax.axis_index("x")
        my_y = lax.axis_index("y")
        my_z = lax.axis_index("z")
        tgt = pi_ref[my_x]

        barrier = pltpu.get_barrier_semaphore()

        @pl.when(tgt == my_x)
        def _():
            out_ref[...] = x_ref[...].astype(out_ref.dtype)

        @pl.when(tgt != my_x)
        def _():
            pl.semaphore_signal(
                barrier,
                inc=1,
                device_id=(tgt, my_y, my_z),
                device_id_type=pl.DeviceIdType.MESH,
            )

            rdmas = []
            for k in range(_N_CHUNKS):
                sl = pl.ds(k * rows, rows)
                send_buf[0, sl, :] = x_ref[0, sl, :].astype(send_buf.dtype)
                if k == 0:
                    pl.semaphore_wait(barrier, 1)
                rdma = pltpu.make_async_remote_copy(
                    src_ref=send_buf.at[:, sl, :],
                    dst_ref=out_ref.at[:, sl, :],
                    send_sem=send_sems.at[k],
                    recv_sem=recv_sems.at[k],
                    device_id=(tgt, my_y, my_z),
                    device_id_type=pl.DeviceIdType.MESH,
                )
                rdma.start()
                rdmas.append(rdma)
            for rdma in rdmas:
                rdma.wait()

    return pl.pallas_call(
        body,
        out_shape=jax.ShapeDtypeStruct(x.shape, jnp.bfloat16),
        in_specs=[
            pl.BlockSpec(memory_space=pltpu.SMEM),
            pl.BlockSpec(memory_space=pltpu.VMEM),
        ],
        out_specs=pl.BlockSpec(memory_space=pltpu.VMEM),
        scratch_shapes=[
            pltpu.VMEM(x.shape, jnp.bfloat16),
            pltpu.SemaphoreType.DMA((_N_CHUNKS,)),
            pltpu.SemaphoreType.DMA((_N_CHUNKS,)),
        ],
        compiler_params=pltpu.CompilerParams(collective_id=0),
    )(pi, x)
